# Initial kernel scaffold; baseline (speedup 1.0000x reference)
import jax
import jax.numpy as jnp
from jax.experimental import pallas as pl


def kernel(x, edge_index, batch, W1a, b1a, W1b, b1b, W2a, b2a, W2b, b2b):
    raise NotImplementedError("write your pallas kernel here")


# speedup vs baseline: 7.4578x; 7.4578x over previous
import functools

import jax
import jax.numpy as jnp
from jax import lax
from jax.experimental import pallas as pl
from jax.experimental.pallas import tpu as pltpu
from jax.experimental.pallas import tpu_sc as plsc

N = 10000
E = 320000
F = 128
F_OUT = 512
NGR = 64
K = 5

NC, NS = 2, 16
NW = NC * NS
CHUNK = 128
CPW = 80
EPW = CHUNK * CPW
EPAD = EPW * NW
NPAD = 10112
RPT = NPAD // NS

_mesh = plsc.VectorSubcoreMesh(core_axis_name="c", subcore_axis_name="s")


@functools.partial(
    pl.kernel,
    out_type=jax.ShapeDtypeStruct((NC, NPAD, F), jnp.float32),
    mesh=_mesh,
    scratch_types=[
        pltpu.VMEM((CPW, CHUNK), jnp.int32),
        pltpu.VMEM((CPW, CHUNK), jnp.int32),
        pltpu.VMEM((CHUNK, F), jnp.float32),
        pltpu.VMEM_SHARED((NPAD, F), jnp.float32),
        pltpu.SemaphoreType.DMA,
    ],
)
def _spmm(u_hbm, src_hbm, dst_hbm, zero_hbm, out_hbm, sidx, didx, rows, acc, sem):
    cid = lax.axis_index("c")
    sid = lax.axis_index("s")
    wid = sid * NC + cid
    r0 = sid * RPT
    pltpu.sync_copy(zero_hbm.at[pl.ds(r0, RPT)], acc.at[pl.ds(r0, RPT)])
    pltpu.sync_copy(src_hbm.at[pl.ds(wid * CPW, CPW)], sidx)
    pltpu.sync_copy(dst_hbm.at[pl.ds(wid * CPW, CPW)], didx)
    plsc.subcore_barrier()

    def body(c, carry):
        pltpu.async_copy(u_hbm.at[sidx.at[c]], rows, sem).wait()
        pltpu.sync_copy(rows, acc.at[didx.at[c]], add=True)
        return carry

    lax.fori_loop(0, CPW, body, 0)

    plsc.subcore_barrier()
    pltpu.sync_copy(acc.at[pl.ds(r0, RPT)], out_hbm.at[cid].at[pl.ds(r0, RPT)])


RB = 400
NRB = N // RB


def _mm_body(x_ref, w_ref, o_ref):
    o_ref[...] = jnp.dot(x_ref[...], w_ref[...], preferred_element_type=jnp.float32)


def _matmul(x, w):
    n, kk = x.shape
    fo = w.shape[1]
    return pl.pallas_call(
        _mm_body,
        grid=(NRB,),
        in_specs=[pl.BlockSpec((RB, kk), lambda i: (i, 0)),
                  pl.BlockSpec((kk, fo), lambda i: (0, 0))],
        out_specs=pl.BlockSpec((RB, fo), lambda i: (i, 0)),
        out_shape=jax.ShapeDtypeStruct((n, fo), jnp.float32),
    )(x, w)


def _pool_body(b_ref, h_ref, s_ref, c_ref):
    g = b_ref[0, 0, :]
    oh = (lax.broadcasted_iota(jnp.int32, (NGR, RB), 0) == g[None, :]).astype(jnp.float32)
    s = jnp.dot(oh, h_ref[...], preferred_element_type=jnp.float32)
    cc = jnp.broadcast_to(jnp.sum(oh, axis=1, keepdims=True), (NGR, 128))

    @pl.when(pl.program_id(0) == 0)
    def _():
        s_ref[...] = jnp.zeros_like(s_ref)
        c_ref[...] = jnp.zeros_like(c_ref)

    s_ref[...] += s
    c_ref[...] += cc


def _pool(batch3, h2):
    return pl.pallas_call(
        _pool_body,
        grid=(NRB,),
        in_specs=[pl.BlockSpec((1, 1, RB), lambda i: (i, 0, 0)),
                  pl.BlockSpec((RB, F_OUT), lambda i: (i, 0))],
        out_specs=[pl.BlockSpec((NGR, F_OUT), lambda i: (0, 0)),
                   pl.BlockSpec((NGR, 128), lambda i: (0, 0))],
        out_shape=[jax.ShapeDtypeStruct((NGR, F_OUT), jnp.float32),
                   jax.ShapeDtypeStruct((NGR, 128), jnp.float32)],
    )(batch3, h2)


def _pad_table(t):
    return jnp.pad(t, ((0, NPAD - N), (0, 0)))


@jax.jit
def kernel(x, edge_index, batch, W1a, b1a, W1b, b1b, W2a, b2a, W2b, b2b):
    row = edge_index[0].astype(jnp.int32)
    col = edge_index[1].astype(jnp.int32)
    pad = (jnp.arange(EPAD - E, dtype=jnp.int32) % 16) + N
    srcA = jnp.concatenate([col, pad]).reshape(NW * CPW, CHUNK)
    dstA = jnp.concatenate([row, pad]).reshape(NW * CPW, CHUNK)
    srcB = jnp.concatenate([row, pad]).reshape(NW * CPW, CHUNK)
    dstB = jnp.concatenate([col, pad]).reshape(NW * CPW, CHUNK)
    zero = jnp.zeros((NPAD, F), jnp.float32)

    def propagate(u_pad, src2d, dst2d):
        out = _spmm(u_pad, src2d, dst2d, zero)
        return (out[0] + out[1])[:N]

    ones_t = jnp.ones((NPAD, F), jnp.float32)
    deg = propagate(ones_t, srcA, dstA)[:, 0]
    dis = jnp.where(deg > 0, lax.rsqrt(jnp.maximum(deg, 1e-12)), 0.0)
    disc = dis[:, None]

    def cheb_basis(v0, src2d, dst2d):
        txs = [v0]
        u = _pad_table(disc * v0)
        for k in range(1, K):
            pv = disc * propagate(u, src2d, dst2d)
            if k == 1:
                tx = -txs[0] / 3.0 - (2.0 / 3.0) * pv
            else:
                tx = -(2.0 / 3.0) * txs[-1] - (4.0 / 3.0) * pv - txs[-2]
            txs.append(tx)
            u = _pad_table(disc * tx)
        return jnp.concatenate(txs, axis=1)

    def layer(v, Wa, ba, Wb, bb):
        Xa = cheb_basis(v, srcA, dstA)
        Xb = cheb_basis(v, srcB, dstB)
        fo = Wa.shape[2]
        oa = _matmul(Xa, Wa.reshape(K * F, fo)) + ba
        ob = _matmul(Xb, Wb.reshape(K * F, fo)) + bb
        return jax.nn.relu(jnp.concatenate([oa, ob], axis=1))

    h = layer(x, W1a, b1a, W1b, b1b)
    h2 = layer(h, W2a, b2a, W2b, b2b)

    batch3 = batch.astype(jnp.int32).reshape(NRB, 1, RB)
    sums, cnts = _pool(batch3, h2)
    return sums / jnp.maximum(cnts[:, :1], 1.0)


# speedup vs baseline: 8.7714x; 1.1761x over previous
import functools

import jax
import jax.numpy as jnp
from jax import lax
from jax.experimental import pallas as pl
from jax.experimental.pallas import tpu as pltpu
from jax.experimental.pallas import tpu_sc as plsc

N = 10000
E = 320000
F = 128
F_OUT = 512
NGR = 64
K = 5

NC, NS = 2, 16
NW = NC * NS
CHUNK = 128
CPW = 80
EPW = CHUNK * CPW
EPAD = EPW * NW
NPAD = 10112
RPT = NPAD // NS

_mesh = plsc.VectorSubcoreMesh(core_axis_name="c", subcore_axis_name="s")
NBUF = 2


def _make_spmm(width):
    @functools.partial(
        pl.kernel,
        out_type=jax.ShapeDtypeStruct((NC, NPAD, width), jnp.float32),
        mesh=_mesh,
        scratch_types=[
            pltpu.VMEM((16, CHUNK), jnp.int32),
            pltpu.VMEM((16, CHUNK), jnp.int32),
            pltpu.VMEM_SHARED((NPAD, width), jnp.float32),
        ] + [pltpu.VMEM((CHUNK, width), jnp.float32) for _ in range(NBUF)]
          + [pltpu.SemaphoreType.DMA for _ in range(NBUF)],
    )
    def _spmm(u_hbm, src_hbm, dst_hbm, zero_hbm, out_hbm, sidx, didx, acc, *bufsems):
        rows = bufsems[:NBUF]
        sems = bufsems[NBUF:]
        cid = lax.axis_index("c")
        sid = lax.axis_index("s")
        wid = sid * NC + cid
        ebase = wid * CPW
        r0 = sid * RPT
        pltpu.sync_copy(zero_hbm.at[pl.ds(r0, RPT)], acc.at[pl.ds(r0, RPT)])
        pltpu.sync_copy(src_hbm.at[pl.ds(ebase, 16)], sidx)
        pltpu.sync_copy(dst_hbm.at[pl.ds(ebase, 16)], didx)
        plsc.subcore_barrier()

        for b in range(NBUF):
            pltpu.async_copy(u_hbm.at[sidx.at[b]], rows[b], sems[b])

        def body(i, carry):
            c0 = i * NBUF

            def restage():
                half = lax.rem(c0 // 8 + 1, 2) * 8
                off = pl.multiple_of(ebase + c0 + 8, 8)
                pltpu.sync_copy(src_hbm.at[pl.ds(off, 8)],
                                sidx.at[pl.ds(half, 8)])
                pltpu.sync_copy(dst_hbm.at[pl.ds(off, 8)],
                                didx.at[pl.ds(half, 8)])

            pl.when(jnp.logical_and(lax.rem(c0, 8) == 0, c0 + 8 < CPW))(restage)

            for b in range(NBUF):
                c = lax.rem(c0 + b, 16)
                pltpu.make_async_copy(u_hbm.at[sidx.at[c]], rows[b], sems[b]).wait()
                pltpu.async_copy(rows[b], acc.at[didx.at[c]], sems[b], add=True)
            for b in range(NBUF):
                c = lax.rem(c0 + b, 16)
                pltpu.make_async_copy(rows[b], acc.at[didx.at[c]], sems[b]).wait()

                def refill(bb=b, cc=c0 + b + NBUF):
                    pltpu.async_copy(u_hbm.at[sidx.at[lax.rem(cc, 16)]],
                                     rows[bb], sems[bb])

                pl.when(c0 + b + NBUF < CPW)(refill)
            return carry

        lax.fori_loop(0, CPW // NBUF, body, 0)

        plsc.subcore_barrier()
        pltpu.sync_copy(acc.at[pl.ds(r0, RPT)], out_hbm.at[cid].at[pl.ds(r0, RPT)])

    return _spmm


_spmm_f = _make_spmm(F)

RB = 400
NRB = N // RB


def _mm_body(x_ref, w_ref, o_ref):
    o_ref[...] = jnp.dot(x_ref[...], w_ref[...], preferred_element_type=jnp.float32)


def _matmul(x, w):
    n, kk = x.shape
    fo = w.shape[1]
    return pl.pallas_call(
        _mm_body,
        grid=(NRB,),
        in_specs=[pl.BlockSpec((RB, kk), lambda i: (i, 0)),
                  pl.BlockSpec((kk, fo), lambda i: (0, 0))],
        out_specs=pl.BlockSpec((RB, fo), lambda i: (i, 0)),
        out_shape=jax.ShapeDtypeStruct((n, fo), jnp.float32),
    )(x, w)


def _pool_body(b_ref, h_ref, s_ref, c_ref):
    g = b_ref[0, 0, :]
    oh = (lax.broadcasted_iota(jnp.int32, (NGR, RB), 0) == g[None, :]).astype(jnp.float32)
    s = jnp.dot(oh, h_ref[...], preferred_element_type=jnp.float32)
    cc = jnp.broadcast_to(jnp.sum(oh, axis=1, keepdims=True), (NGR, 128))

    @pl.when(pl.program_id(0) == 0)
    def _():
        s_ref[...] = jnp.zeros_like(s_ref)
        c_ref[...] = jnp.zeros_like(c_ref)

    s_ref[...] += s
    c_ref[...] += cc


def _pool(batch3, h2):
    return pl.pallas_call(
        _pool_body,
        grid=(NRB,),
        in_specs=[pl.BlockSpec((1, 1, RB), lambda i: (i, 0, 0)),
                  pl.BlockSpec((RB, F_OUT), lambda i: (i, 0))],
        out_specs=[pl.BlockSpec((NGR, F_OUT), lambda i: (0, 0)),
                   pl.BlockSpec((NGR, 128), lambda i: (0, 0))],
        out_shape=[jax.ShapeDtypeStruct((NGR, F_OUT), jnp.float32),
                   jax.ShapeDtypeStruct((NGR, 128), jnp.float32)],
    )(batch3, h2)


def _pad_table(t):
    return jnp.pad(t, ((0, NPAD - N), (0, 0)))


@jax.jit
def kernel(x, edge_index, batch, W1a, b1a, W1b, b1b, W2a, b2a, W2b, b2b):
    row = edge_index[0].astype(jnp.int32)
    col = edge_index[1].astype(jnp.int32)
    pad = (jnp.arange(EPAD - E, dtype=jnp.int32) % 16) + N
    srcA = jnp.concatenate([col, pad]).reshape(NW * CPW, CHUNK)
    dstA = jnp.concatenate([row, pad]).reshape(NW * CPW, CHUNK)
    srcB = jnp.concatenate([row, pad]).reshape(NW * CPW, CHUNK)
    dstB = jnp.concatenate([col, pad]).reshape(NW * CPW, CHUNK)
    zero = jnp.zeros((NPAD, F), jnp.float32)

    def propagate(u_pad, src2d, dst2d):
        out = _spmm_f(u_pad, src2d, dst2d, zero)
        return (out[0] + out[1])[:N]

    ones_t = jnp.ones((NPAD, F), jnp.float32)
    deg = propagate(ones_t, srcA, dstA)[:, 0]
    dis = jnp.where(deg > 0, lax.rsqrt(jnp.maximum(deg, 1e-12)), 0.0)
    disc = dis[:, None]

    def cheb_basis(v0, src2d, dst2d):
        txs = [v0]
        u = _pad_table(disc * v0)
        for k in range(1, K):
            pv = disc * propagate(u, src2d, dst2d)
            if k == 1:
                tx = -txs[0] / 3.0 - (2.0 / 3.0) * pv
            else:
                tx = -(2.0 / 3.0) * txs[-1] - (4.0 / 3.0) * pv - txs[-2]
            txs.append(tx)
            u = _pad_table(disc * tx)
        return jnp.concatenate(txs, axis=1)

    def layer(v, Wa, ba, Wb, bb):
        Xa = cheb_basis(v, srcA, dstA)
        Xb = cheb_basis(v, srcB, dstB)
        fo = Wa.shape[2]
        oa = _matmul(Xa, Wa.reshape(K * F, fo)) + ba
        ob = _matmul(Xb, Wb.reshape(K * F, fo)) + bb
        return jax.nn.relu(jnp.concatenate([oa, ob], axis=1))

    h = layer(x, W1a, b1a, W1b, b1b)
    h2 = layer(h, W2a, b2a, W2b, b2b)

    batch3 = batch.astype(jnp.int32).reshape(NRB, 1, RB)
    sums, cnts = _pool(batch3, h2)
    return sums / jnp.maximum(cnts[:, :1], 1.0)


# speedup vs baseline: 9.5765x; 1.0918x over previous
import functools

import jax
import jax.numpy as jnp
from jax import lax
from jax.experimental import pallas as pl
from jax.experimental.pallas import tpu as pltpu
from jax.experimental.pallas import tpu_sc as plsc

N = 10000
E = 320000
F = 128
F_OUT = 512
NGR = 64
K = 5

NC, NS = 2, 16
NW = NC * NS
CHUNK = 128
CPW = 80
EPW = CHUNK * CPW
EPAD = EPW * NW
NPAD = 10112
RPT = NPAD // NS

_mesh = plsc.VectorSubcoreMesh(core_axis_name="c", subcore_axis_name="s")
NBUF = 2


def _make_spmm(width):
    @functools.partial(
        pl.kernel,
        out_type=jax.ShapeDtypeStruct((NC, NPAD, width), jnp.float32),
        mesh=_mesh,
        scratch_types=[
            pltpu.VMEM((16, CHUNK), jnp.int32),
            pltpu.VMEM((16, CHUNK), jnp.int32),
            pltpu.VMEM_SHARED((NPAD, width), jnp.float32),
        ] + [pltpu.VMEM((CHUNK, width), jnp.float32) for _ in range(NBUF)]
          + [pltpu.SemaphoreType.DMA for _ in range(NBUF)],
    )
    def _spmm(u_hbm, src_hbm, dst_hbm, zero_hbm, out_hbm, sidx, didx, acc, *bufsems):
        rows = bufsems[:NBUF]
        sems = bufsems[NBUF:]
        cid = lax.axis_index("c")
        sid = lax.axis_index("s")
        wid = sid * NC + cid
        ebase = wid * CPW
        r0 = sid * RPT
        pltpu.sync_copy(zero_hbm.at[pl.ds(r0, RPT)], acc.at[pl.ds(r0, RPT)])
        pltpu.sync_copy(src_hbm.at[pl.ds(ebase, 16)], sidx)
        pltpu.sync_copy(dst_hbm.at[pl.ds(ebase, 16)], didx)
        plsc.subcore_barrier()

        def gather(c, b):
            pltpu.async_copy(u_hbm.at[sidx.at[lax.rem(c, 16)]], rows[b], sems[b])

        def wait_gather(c, b):
            pltpu.make_async_copy(u_hbm.at[sidx.at[lax.rem(c, 16)]],
                                  rows[b], sems[b]).wait()

        def scatter(c, b):
            pltpu.async_copy(rows[b], acc.at[didx.at[lax.rem(c, 16)]],
                             sems[b], add=True)

        def wait_scatter(c, b):
            pltpu.make_async_copy(rows[b], acc.at[didx.at[lax.rem(c, 16)]],
                                  sems[b]).wait()

        gather(0, 0)

        def body(i, carry):
            c0 = 2 * i
            c1 = c0 + 1

            def restage():
                half = lax.rem(c0 // 8 + 1, 2) * 8
                off = pl.multiple_of(ebase + c0 + 8, 8)
                pltpu.sync_copy(src_hbm.at[pl.ds(off, 8)],
                                sidx.at[pl.ds(half, 8)])
                pltpu.sync_copy(dst_hbm.at[pl.ds(off, 8)],
                                didx.at[pl.ds(half, 8)])

            pl.when(jnp.logical_and(lax.rem(c0, 8) == 0, c0 + 8 < CPW))(restage)

            wait_gather(c0, 0)
            scatter(c0, 0)
            pl.when(i > 0)(lambda: wait_scatter(c0 - 1, 1))
            gather(c1, 1)
            wait_gather(c1, 1)
            scatter(c1, 1)
            wait_scatter(c0, 0)
            pl.when(c1 + 1 < CPW)(lambda: gather(c1 + 1, 0))
            return carry

        lax.fori_loop(0, CPW // 2, body, 0)
        wait_scatter(CPW - 1, 1)

        plsc.subcore_barrier()
        pltpu.sync_copy(acc.at[pl.ds(r0, RPT)], out_hbm.at[cid].at[pl.ds(r0, RPT)])

    return _spmm


_spmm_f = _make_spmm(F)

RB = 400
NRB = N // RB


def _mm_body(x_ref, w_ref, o_ref):
    o_ref[...] = jnp.dot(x_ref[...], w_ref[...], preferred_element_type=jnp.float32)


def _matmul(x, w):
    n, kk = x.shape
    fo = w.shape[1]
    return pl.pallas_call(
        _mm_body,
        grid=(NRB,),
        in_specs=[pl.BlockSpec((RB, kk), lambda i: (i, 0)),
                  pl.BlockSpec((kk, fo), lambda i: (0, 0))],
        out_specs=pl.BlockSpec((RB, fo), lambda i: (i, 0)),
        out_shape=jax.ShapeDtypeStruct((n, fo), jnp.float32),
    )(x, w)


def _pool_body(b_ref, h_ref, s_ref, c_ref):
    g = b_ref[0, 0, :]
    oh = (lax.broadcasted_iota(jnp.int32, (NGR, RB), 0) == g[None, :]).astype(jnp.float32)
    s = jnp.dot(oh, h_ref[...], preferred_element_type=jnp.float32)
    cc = jnp.broadcast_to(jnp.sum(oh, axis=1, keepdims=True), (NGR, 128))

    @pl.when(pl.program_id(0) == 0)
    def _():
        s_ref[...] = jnp.zeros_like(s_ref)
        c_ref[...] = jnp.zeros_like(c_ref)

    s_ref[...] += s
    c_ref[...] += cc


def _pool(batch3, h2):
    return pl.pallas_call(
        _pool_body,
        grid=(NRB,),
        in_specs=[pl.BlockSpec((1, 1, RB), lambda i: (i, 0, 0)),
                  pl.BlockSpec((RB, F_OUT), lambda i: (i, 0))],
        out_specs=[pl.BlockSpec((NGR, F_OUT), lambda i: (0, 0)),
                   pl.BlockSpec((NGR, 128), lambda i: (0, 0))],
        out_shape=[jax.ShapeDtypeStruct((NGR, F_OUT), jnp.float32),
                   jax.ShapeDtypeStruct((NGR, 128), jnp.float32)],
    )(batch3, h2)


def _pad_table(t):
    return jnp.pad(t, ((0, NPAD - N), (0, 0)))


@jax.jit
def kernel(x, edge_index, batch, W1a, b1a, W1b, b1b, W2a, b2a, W2b, b2b):
    row = edge_index[0].astype(jnp.int32)
    col = edge_index[1].astype(jnp.int32)
    pad = (jnp.arange(EPAD - E, dtype=jnp.int32) % 16) + N
    srcA = jnp.concatenate([col, pad]).reshape(NW * CPW, CHUNK)
    dstA = jnp.concatenate([row, pad]).reshape(NW * CPW, CHUNK)
    srcB = jnp.concatenate([row, pad]).reshape(NW * CPW, CHUNK)
    dstB = jnp.concatenate([col, pad]).reshape(NW * CPW, CHUNK)
    zero = jnp.zeros((NPAD, F), jnp.float32)

    def propagate(u_pad, src2d, dst2d):
        out = _spmm_f(u_pad, src2d, dst2d, zero)
        return (out[0] + out[1])[:N]

    ones_t = jnp.ones((NPAD, F), jnp.float32)
    deg = propagate(ones_t, srcA, dstA)[:, 0]
    dis = jnp.where(deg > 0, lax.rsqrt(jnp.maximum(deg, 1e-12)), 0.0)
    disc = dis[:, None]

    def cheb_basis(v0, src2d, dst2d):
        txs = [v0]
        u = _pad_table(disc * v0)
        for k in range(1, K):
            pv = disc * propagate(u, src2d, dst2d)
            if k == 1:
                tx = -txs[0] / 3.0 - (2.0 / 3.0) * pv
            else:
                tx = -(2.0 / 3.0) * txs[-1] - (4.0 / 3.0) * pv - txs[-2]
            txs.append(tx)
            u = _pad_table(disc * tx)
        return jnp.concatenate(txs, axis=1)

    def layer(v, Wa, ba, Wb, bb):
        Xa = cheb_basis(v, srcA, dstA)
        Xb = cheb_basis(v, srcB, dstB)
        fo = Wa.shape[2]
        oa = _matmul(Xa, Wa.reshape(K * F, fo)) + ba
        ob = _matmul(Xb, Wb.reshape(K * F, fo)) + bb
        return jax.nn.relu(jnp.concatenate([oa, ob], axis=1))

    h = layer(x, W1a, b1a, W1b, b1b)
    h2 = layer(h, W2a, b2a, W2b, b2b)

    batch3 = batch.astype(jnp.int32).reshape(NRB, 1, RB)
    sums, cnts = _pool(batch3, h2)
    return sums / jnp.maximum(cnts[:, :1], 1.0)
